# Initial kernel scaffold; baseline (speedup 1.0000x reference)
import jax
import jax.numpy as jnp
from jax.experimental import pallas as pl


def kernel(features, label, camid):
    raise NotImplementedError("write your pallas kernel here")


# speedup vs baseline: 28.1687x; 28.1687x over previous
import jax
import jax.numpy as jnp
from jax.experimental import pallas as pl

_TEMP = 0.1
_N = 1024
_M = 2 * _N
_D = 256
_BLK = 256


def _row_loss(lg, pos, neg):
    neg_l = jnp.where(neg, lg, -1e30)
    m_neg = jnp.max(neg_l, axis=1, keepdims=True)
    s = jnp.sum(jnp.where(neg, jnp.exp(neg_l - m_neg), 0.0), axis=1,
                keepdims=True)
    m = jnp.maximum(lg, m_neg)
    lse = m + jnp.log(jnp.exp(lg - m) + jnp.exp(m_neg - m) * s)
    per_pos = lse - lg
    cnt = jnp.sum(pos.astype(jnp.float32), axis=1)
    return jnp.sum(jnp.where(pos, per_pos, 0.0), axis=1) / cnt


def _scl_kernel(lab_all_ref, cam_all_ref, lab_col_ref, cam_col_ref,
                frow_ref, fall_ref, out_ref):
    i = pl.program_id(0)
    lg = jax.lax.dot_general(
        frow_ref[...], fall_ref[...],
        dimension_numbers=(((1,), (1,)), ((), ())),
        preferred_element_type=jnp.float32) * (1.0 / _TEMP)

    lab_all = lab_all_ref[...]
    cam_all = cam_all_ref[...]
    lab_row = lab_col_ref[...]
    cam_row = cam_col_ref[...]

    eq_lab = lab_row == lab_all
    eq_pair = eq_lab & (cam_row == cam_all)
    row_g = jax.lax.broadcasted_iota(jnp.int32, (_BLK, _M), 0) + i * _BLK
    col_g = jax.lax.broadcasted_iota(jnp.int32, (_BLK, _M), 1)
    offdiag = row_g != col_g

    row_id = _row_loss(lg, eq_lab & offdiag, ~eq_lab)
    row_cam = _row_loss(lg, eq_pair & offdiag, ~eq_pair)
    blk_sum = (jnp.sum(row_id + 0.5 * row_cam) * (1.0 / _M)).reshape(1, 1)

    @pl.when(i == 0)
    def _():
        out_ref[...] = jnp.zeros_like(out_ref)

    out_ref[...] += blk_sum


def kernel(features, label, camid):
    f = jnp.concatenate([features[:, 0, :], features[:, 1, :]], axis=0)
    lab2 = jnp.concatenate([label, label])
    cam2 = jnp.concatenate([camid, camid])
    lab_all = lab2.reshape(1, _M)
    cam_all = cam2.reshape(1, _M)
    lab_col = lab2.reshape(_M, 1)
    cam_col = cam2.reshape(_M, 1)

    grid = _M // _BLK
    out = pl.pallas_call(
        _scl_kernel,
        grid=(grid,),
        in_specs=[
            pl.BlockSpec((1, _M), lambda i: (0, 0)),
            pl.BlockSpec((1, _M), lambda i: (0, 0)),
            pl.BlockSpec((_BLK, 1), lambda i: (i, 0)),
            pl.BlockSpec((_BLK, 1), lambda i: (i, 0)),
            pl.BlockSpec((_BLK, _D), lambda i: (i, 0)),
            pl.BlockSpec((_M, _D), lambda i: (0, 0)),
        ],
        out_specs=pl.BlockSpec((1, 1), lambda i: (0, 0)),
        out_shape=jax.ShapeDtypeStruct((1, 1), jnp.float32),
    )(lab_all, cam_all, lab_col, cam_col, f, f)
    return out.reshape(())


# speedup vs baseline: 28.8658x; 1.0247x over previous
import jax
import jax.numpy as jnp
from jax.experimental import pallas as pl

_TEMP = 0.1
_N = 1024
_M = 2 * _N
_D = 256
_BLK = 256


def _family_loss(lg, m_all, s, pos):
    x = (m_all + jnp.log(s)) - lg
    sp = jnp.maximum(x, 0.0) + jnp.log1p(jnp.exp(-jnp.abs(x)))
    cnt = jnp.sum(pos.astype(jnp.float32), axis=1)
    return jnp.sum(jnp.where(pos, sp, 0.0), axis=1) / cnt


def _scl_kernel(lab_all_ref, cam_all_ref, lab_col_ref, cam_col_ref,
                frow_ref, fall_ref, out_ref):
    i = pl.program_id(0)
    lg = jax.lax.dot_general(
        frow_ref[...], fall_ref[...],
        dimension_numbers=(((1,), (1,)), ((), ())),
        preferred_element_type=jnp.float32) * (1.0 / _TEMP)

    lab_all = lab_all_ref[...]
    cam_all = cam_all_ref[...]
    lab_row = lab_col_ref[...]
    cam_row = cam_col_ref[...]

    eq_lab = lab_row == lab_all
    eq_pair = eq_lab & (cam_row == cam_all)
    row_g = jax.lax.broadcasted_iota(jnp.int32, (_BLK, _M), 0) + i * _BLK
    col_g = jax.lax.broadcasted_iota(jnp.int32, (_BLK, _M), 1)
    offdiag = row_g != col_g

    neg_cam = ~eq_pair
    neg_id = ~eq_lab
    m_all = jnp.max(jnp.where(neg_cam, lg, -1e30), axis=1, keepdims=True)
    e = jnp.exp(lg - m_all)
    s_id = jnp.sum(jnp.where(neg_id, e, 0.0), axis=1, keepdims=True)
    s_cam = s_id + jnp.sum(jnp.where(eq_lab & neg_cam, e, 0.0), axis=1,
                           keepdims=True)

    row_id = _family_loss(lg, m_all, s_id, eq_lab & offdiag)
    row_cam = _family_loss(lg, m_all, s_cam, eq_pair & offdiag)
    blk_sum = (jnp.sum(row_id + 0.5 * row_cam) * (1.0 / _M)).reshape(1, 1)

    @pl.when(i == 0)
    def _():
        out_ref[...] = jnp.zeros_like(out_ref)

    out_ref[...] += blk_sum


def kernel(features, label, camid):
    f = jnp.concatenate([features[:, 0, :], features[:, 1, :]], axis=0)
    lab2 = jnp.concatenate([label, label])
    cam2 = jnp.concatenate([camid, camid])
    lab_all = lab2.reshape(1, _M)
    cam_all = cam2.reshape(1, _M)
    lab_col = lab2.reshape(_M, 1)
    cam_col = cam2.reshape(_M, 1)

    grid = _M // _BLK
    out = pl.pallas_call(
        _scl_kernel,
        grid=(grid,),
        in_specs=[
            pl.BlockSpec((1, _M), lambda i: (0, 0)),
            pl.BlockSpec((1, _M), lambda i: (0, 0)),
            pl.BlockSpec((_BLK, 1), lambda i: (i, 0)),
            pl.BlockSpec((_BLK, 1), lambda i: (i, 0)),
            pl.BlockSpec((_BLK, _D), lambda i: (i, 0)),
            pl.BlockSpec((_M, _D), lambda i: (0, 0)),
        ],
        out_specs=pl.BlockSpec((1, 1), lambda i: (0, 0)),
        out_shape=jax.ShapeDtypeStruct((1, 1), jnp.float32),
    )(lab_all, cam_all, lab_col, cam_col, f, f)
    return out.reshape(())


# speedup vs baseline: 33.5454x; 1.1621x over previous
import jax
import jax.numpy as jnp
from jax.experimental import pallas as pl

_TEMP = 0.1
_N = 1024
_M = 2 * _N
_D = 256
_BLK = 256


def _softplus(x):
    xc = jnp.minimum(x, 80.0)
    return jnp.log1p(jnp.exp(xc)) + jnp.maximum(x - 80.0, 0.0)


def _scl_kernel(lab_all_ref, key_all_ref, lab_col_ref, key_col_ref,
                frow_ref, fall_ref, out_ref):
    i = pl.program_id(0)
    frow = frow_ref[...]
    lg = jax.lax.dot_general(
        frow, fall_ref[...],
        dimension_numbers=(((1,), (1,)), ((), ())),
        preferred_element_type=jnp.float32)

    eq_lab = lab_col_ref[...] == lab_all_ref[...]
    eq_pair = key_col_ref[...] == key_all_ref[...]

    m_all = jnp.max(jnp.where(eq_pair, -1e30, lg), axis=1, keepdims=True)
    e = jnp.exp(lg - m_all)
    s_id = jnp.sum(jnp.where(eq_lab, 0.0, e), axis=1, keepdims=True)
    s_cam = jnp.sum(jnp.where(eq_pair, 0.0, e), axis=1, keepdims=True)
    log_s_id = m_all + jnp.log(s_id)
    log_s_cam = m_all + jnp.log(s_cam)

    lg_diag = jnp.sum(frow * frow, axis=1, keepdims=True)

    cnt_id = jnp.sum(eq_lab, axis=1).astype(jnp.float32) - 1.0
    cnt_cam = jnp.sum(eq_pair, axis=1).astype(jnp.float32) - 1.0

    sp_id = _softplus(log_s_id - lg)
    sp_cam = _softplus(log_s_cam - lg)
    t_id = jnp.sum(jnp.where(eq_lab, sp_id, 0.0), axis=1)
    t_cam = jnp.sum(jnp.where(eq_pair, sp_cam, 0.0), axis=1)
    row_id = (t_id - _softplus(log_s_id - lg_diag)[:, 0]) / cnt_id
    row_cam = (t_cam - _softplus(log_s_cam - lg_diag)[:, 0]) / cnt_cam

    blk_sum = (jnp.sum(row_id + 0.5 * row_cam) * (1.0 / _M)).reshape(1, 1)

    @pl.when(i == 0)
    def _():
        out_ref[...] = jnp.zeros_like(out_ref)

    out_ref[...] += blk_sum


def kernel(features, label, camid):
    f = features.reshape(_M, _D) * jnp.sqrt(jnp.float32(1.0 / _TEMP))
    lab2 = jnp.repeat(label, 2)
    key2 = lab2 * 8 + jnp.repeat(camid, 2)
    lab_all = lab2.reshape(1, _M)
    key_all = key2.reshape(1, _M)
    lab_col = lab2.reshape(_M, 1)
    key_col = key2.reshape(_M, 1)

    grid = _M // _BLK
    out = pl.pallas_call(
        _scl_kernel,
        grid=(grid,),
        in_specs=[
            pl.BlockSpec((1, _M), lambda i: (0, 0)),
            pl.BlockSpec((1, _M), lambda i: (0, 0)),
            pl.BlockSpec((_BLK, 1), lambda i: (i, 0)),
            pl.BlockSpec((_BLK, 1), lambda i: (i, 0)),
            pl.BlockSpec((_BLK, _D), lambda i: (i, 0)),
            pl.BlockSpec((_M, _D), lambda i: (0, 0)),
        ],
        out_specs=pl.BlockSpec((1, 1), lambda i: (0, 0)),
        out_shape=jax.ShapeDtypeStruct((1, 1), jnp.float32),
    )(lab_all, key_all, lab_col, key_col, f, f)
    return out.reshape(())
